# baseline (speedup 1.0000x reference)
import jax
import jax.numpy as jnp
import numpy as np
from jax import lax
from jax.experimental import pallas as pl
from jax.experimental.pallas import tpu as pltpu
from jax.experimental.pallas import tpu_sc as plsc

N_POINTS = 524288
DIM = 3
N_LEVELS = 16
F = 2
LOG2_T = 19
T = 2 ** LOG2_T
BASE_RES = 16
PER_LEVEL_SCALE = 1.5

NC = 2
NS = 16
NW = NC * NS
L = 16

PW = N_POINTS // NW
C = 128
NCHUNK = PW // C
LPS = 4
NSET = N_LEVELS // LPS
NSTREAM = LPS * 8
TW = N_LEVELS * T * F
NROWS8 = TW // 8

WPW = TW // NW
RCH = 16384
NRCH = WPW // RCH

_P1 = np.int32(np.uint32(2654435761).astype(np.int32))
_P2 = np.int32(np.uint32(805459861).astype(np.int32))
_MASK = np.int32(T - 1)

_RES = [int(np.floor(BASE_RES * (PER_LEVEL_SCALE ** l))) for l in range(N_LEVELS)]
_DENSE = [(r + 1) ** DIM <= T for r in _RES]


def _iota16():
    return lax.broadcasted_iota(jnp.int32, (L,), 0)


def _relayout_body(tn_ref, t8_ref, ibuf0, ibuf1, obuf0, obuf1,
                   semi0, semi1, semo0, semo1):
    wid = lax.axis_index("c") * NS + lax.axis_index("s")
    wbase = wid * WPW

    ibuf = (ibuf0, ibuf1)
    obuf = (obuf0, obuf1)
    semi = (semi0, semi1)
    semo = (semo0, semo1)

    def stage_in(ci, s):
        pltpu.async_copy(tn_ref.at[pl.ds(wbase + ci * RCH, RCH)],
                         ibuf[s], semi[s])

    oidx0 = 2 * _iota16()

    def interleave(s):
        def blk(b, _):
            o = b * 256
            for k in range(8):
                v0 = ibuf[s][pl.ds(o + k * 16, L)]
                v1 = ibuf[s][pl.ds(o + 128 + k * 16, L)]
                tgt = oidx0 + (o + k * 32)
                plsc.store_scatter(obuf[s], [tgt], v0)
                plsc.store_scatter(obuf[s], [tgt + 1], v1)
            return 0

        lax.fori_loop(0, RCH // 256, blk, 0, unroll=False)

    stage_in(jnp.int32(0), 0)

    def chunk(i2, _):
        for half in range(2):
            ci = 2 * i2 + half
            nxt = jnp.minimum(ci + 1, NRCH - 1)
            stage_in(nxt, (half + 1) % 2)
            pltpu.make_async_copy(
                tn_ref.at[pl.ds(wbase, RCH)], ibuf[half], semi[half]).wait()
            @pl.when(ci >= 2)
            def _():
                pltpu.make_async_copy(
                    obuf[half], t8_ref.at[pl.ds(wbase, RCH)],
                    semo[half]).wait()
            interleave(half)
            pltpu.async_copy(
                obuf[half], t8_ref.at[pl.ds(wbase + ci * RCH, RCH)],
                semo[half])
        return 0

    lax.fori_loop(0, NRCH // 2, chunk, 0, unroll=False)
    pltpu.make_async_copy(tn_ref.at[pl.ds(wbase, RCH)], ibuf[0], semi[0]).wait()
    pltpu.make_async_copy(obuf[0], t8_ref.at[pl.ds(wbase, RCH)], semo[0]).wait()
    pltpu.make_async_copy(obuf[1], t8_ref.at[pl.ds(wbase, RCH)], semo[1]).wait()


def _phase_a(xs, idx_s, lo_s, w_s, st):
    def group(g, _):
        ii = _iota16()
        p3 = (g * L + ii) * 3
        px = plsc.load_gather(xs, [p3])
        py = plsc.load_gather(xs, [p3 + 1])
        pz = plsc.load_gather(xs, [p3 + 2])
        sl = pl.ds(g * L, L)
        for lv in range(LPS):
            lvl = st * LPS + lv
            res = _RES[lvl]
            rf = jnp.float32(res)
            pox, poy, poz = px * rf, py * rf, pz * rf
            ix = pox.astype(jnp.int32)
            iy = poy.astype(jnp.int32)
            iz = poz.astype(jnp.int32)
            fx = pox - ix.astype(jnp.float32)
            fy = poy - iy.astype(jnp.float32)
            fz = poz - iz.astype(jnp.float32)
            if _DENSE[lvl]:
                stride = jnp.int32(res + 1)
                stride2 = jnp.int32((res + 1) * (res + 1))
                ax0, ax1 = ix, ix + 1
                by0 = iy * stride
                by1 = by0 + stride
                cz0 = iz * stride2 + jnp.int32(lvl * T)
                cz1 = cz0 + stride2
                xy = [ax0 + by0, ax1 + by0, ax0 + by1, ax1 + by1]
                for c in range(8):
                    idx = xy[c & 3] + (cz0 if c < 4 else cz1)
                    r = lv * 8 + c
                    idx_s[r, sl] = idx >> 2
                    lo_s[r, sl] = (idx & 3) * 2
            else:
                ax0, ax1 = ix, ix + 1
                by0 = iy * _P1
                by1 = by0 + _P1
                cz0 = iz * _P2
                cz1 = cz0 + _P2
                off4 = jnp.int32(lvl * (T // 4))
                m4 = jnp.int32((T - 1) >> 2)
                xy = [ax0 ^ by0, ax1 ^ by0, ax0 ^ by1, ax1 ^ by1]
                for c in range(8):
                    h = xy[c & 3] ^ (cz0 if c < 4 else cz1)
                    r = lv * 8 + c
                    idx_s[r, sl] = ((h >> 2) & m4) + off4
                    lo_s[r, sl] = (h & 3) * 2
            wx1, wy1, wz1 = fx, fy, fz
            wx0, wy0, wz0 = 1.0 - fx, 1.0 - fy, 1.0 - fz
            wxy = [wx0 * wy0, wx1 * wy0, wx0 * wy1, wx1 * wy1]
            for c in range(8):
                w_s[lv * 8 + c, sl] = wxy[c & 3] * (wz0 if c < 4 else wz1)
        return 0

    lax.fori_loop(0, C // L, group, 0, unroll=False)


def _fire(t_ref, idx_s, rows_s, sem):
    def body(j, _):
        pltpu.async_copy(t_ref.at[idx_s.at[j]], rows_s.at[j], sem)
        return 0

    lax.fori_loop(0, NSTREAM, body, 0, unroll=False)


def _drain(t_ref, idx_s, rows_s, sem):
    def body(j, _):
        pltpu.make_async_copy(t_ref.at[idx_s.at[j]], rows_s.at[j], sem).wait()
        return 0

    lax.fori_loop(0, NSTREAM, body, 0, unroll=False)


def _phase_b(lo_s, w_s, rows_s, out_v, st):
    def group(g, _):
        ii = _iota16()
        pt = g * L + ii
        sl = pl.ds(g * L, L)
        for lv in range(LPS):
            acc0 = None
            acc1 = None
            for c in range(8):
                j = lv * 8 + c
                jv = jnp.full((L,), j, jnp.int32)
                wv = w_s[j, sl]
                lo = lo_s[j, sl]
                v0 = plsc.load_gather(rows_s, [jv, pt, lo])
                v1 = plsc.load_gather(rows_s, [jv, pt, lo + 1])
                if acc0 is None:
                    acc0, acc1 = wv * v0, wv * v1
                else:
                    acc0 = acc0 + wv * v0
                    acc1 = acc1 + wv * v1
            out_v[st, pl.ds(2 * lv * 128 + g * L, L)] = acc0
            out_v[st, pl.ds(2 * lv * 128 + 128 + g * L, L)] = acc1
        return 0

    lax.fori_loop(0, C // L, group, 0, unroll=False)


def _body(x_ref, t_ref, out_ref,
          xs0, xs1, idx0, idx1, lo0, lo1, w0, w1, rows0, rows1, out_v,
          sem0, sem1, semo):
    wid = lax.axis_index("c") * NS + lax.axis_index("s")
    pbase = wid * PW

    xs = (xs0, xs1)
    idx = (idx0, idx1)
    lo = (lo0, lo1)
    wb = (w0, w1)
    rows = (rows0, rows1)
    sem = (sem0, sem1)

    def fire_unit(ci, st, s):
        base = pbase + ci * C
        pltpu.sync_copy(x_ref.at[pl.ds(base * 3, C * 3)], xs[s])
        _phase_a(xs[s], idx[s], lo[s], wb[s], st)
        _fire(t_ref, idx[s], rows[s], sem[s])

    def out_dst(i):
        return out_ref.at[:, pl.ds((wid * NCHUNK + i) * 1024, 1024)]

    fire_unit(jnp.int32(0), 0, 0)

    def chunk(i, _):
        for st in range(NSET):
            if st < NSET - 1:
                fire_unit(i, st + 1, (st + 1) % 2)
            else:
                fire_unit(jnp.minimum(i + 1, NCHUNK - 1), 0, 0)
            _drain(t_ref, idx[st % 2], rows[st % 2], sem[st % 2])
            if st == 0:
                @pl.when(i >= 1)
                def _():
                    pltpu.make_async_copy(out_v, out_dst(i - 1), semo).wait()
            _phase_b(lo[st % 2], wb[st % 2], rows[st % 2], out_v, st)
        pltpu.async_copy(out_v, out_dst(i), semo)
        return 0

    lax.fori_loop(0, NCHUNK, chunk, 0, unroll=False)
    _drain(t_ref, idx[0], rows[0], sem[0])
    pltpu.make_async_copy(out_v, out_dst(NCHUNK - 1), semo).wait()


@jax.jit
def _grid_encode(x_flat, tn):
    mesh = plsc.VectorSubcoreMesh(core_axis_name="c", subcore_axis_name="s")
    cp = pltpu.CompilerParams(needs_layout_passes=False,
                              use_tc_tiling_on_sc=False)

    relayout = pl.kernel(
        _relayout_body,
        out_type=jax.ShapeDtypeStruct((TW,), jnp.float32),
        mesh=mesh,
        scratch_types=[
            pltpu.VMEM((RCH,), jnp.float32),
            pltpu.VMEM((RCH,), jnp.float32),
            pltpu.VMEM((RCH,), jnp.float32),
            pltpu.VMEM((RCH,), jnp.float32),
            pltpu.SemaphoreType.DMA,
            pltpu.SemaphoreType.DMA,
            pltpu.SemaphoreType.DMA,
            pltpu.SemaphoreType.DMA,
        ],
        compiler_params=cp,
    )
    t8 = relayout(tn).reshape(NROWS8, 8)

    scratch = [
        pltpu.VMEM((C * 3,), jnp.float32),
        pltpu.VMEM((C * 3,), jnp.float32),
        pltpu.VMEM((NSTREAM, C), jnp.int32),
        pltpu.VMEM((NSTREAM, C), jnp.int32),
        pltpu.VMEM((NSTREAM, C), jnp.int32),
        pltpu.VMEM((NSTREAM, C), jnp.int32),
        pltpu.VMEM((NSTREAM, C), jnp.float32),
        pltpu.VMEM((NSTREAM, C), jnp.float32),
        pltpu.VMEM((NSTREAM, C, 8), jnp.float32),
        pltpu.VMEM((NSTREAM, C, 8), jnp.float32),
        pltpu.VMEM((4, 1024), jnp.float32),
        pltpu.SemaphoreType.DMA,
        pltpu.SemaphoreType.DMA,
        pltpu.SemaphoreType.DMA,
    ]
    run = pl.kernel(
        _body,
        out_type=jax.ShapeDtypeStruct((4, N_POINTS * 8), jnp.float32),
        mesh=mesh,
        scratch_types=scratch,
        compiler_params=cp,
    )
    return run(x_flat, t8)


def kernel(x, table):
    tn = (table.reshape(N_LEVELS, T // 128, 128, F)
          .transpose(0, 1, 3, 2)
          .reshape(TW))
    out = _grid_encode(x.reshape(-1), tn)
    return out


# speedup vs baseline: 1.7664x; 1.7664x over previous
import jax
import jax.numpy as jnp
import numpy as np
from jax import lax
from jax.experimental import pallas as pl
from jax.experimental.pallas import tpu as pltpu
from jax.experimental.pallas import tpu_sc as plsc

N_POINTS = 524288
DIM = 3
N_LEVELS = 16
F = 2
LOG2_T = 19
T = 2 ** LOG2_T
BASE_RES = 16
PER_LEVEL_SCALE = 1.5

NC = 2
NS = 16
NW = NC * NS
L = 16

PW = N_POINTS // NW
C = 128
NCHUNK = PW // C
LPS = 4
NSET = N_LEVELS // LPS
NSTREAM = LPS * 8
TW = N_LEVELS * T * F
NROWS8 = TW // 8

WPW = TW // NW
RCH = 16384
NRCH = WPW // RCH

_P1 = np.int32(np.uint32(2654435761).astype(np.int32))
_P2 = np.int32(np.uint32(805459861).astype(np.int32))
_MASK = np.int32(T - 1)

_RES = [int(np.floor(BASE_RES * (PER_LEVEL_SCALE ** l))) for l in range(N_LEVELS)]
_DENSE = [(r + 1) ** DIM <= T for r in _RES]


def _iota16():
    return lax.broadcasted_iota(jnp.int32, (L,), 0)


def _relayout_body(tn_ref, t8_ref, ibuf0, ibuf1, obuf0, obuf1,
                   semi0, semi1, semo0, semo1):
    wid = lax.axis_index("c") * NS + lax.axis_index("s")
    wbase = wid * WPW

    ibuf = (ibuf0, ibuf1)
    obuf = (obuf0, obuf1)
    semi = (semi0, semi1)
    semo = (semo0, semo1)

    def stage_in(ci, s):
        pltpu.async_copy(tn_ref.at[pl.ds(wbase + ci * RCH, RCH)],
                         ibuf[s], semi[s])

    oidx0 = 2 * _iota16()

    def interleave(s):
        def blk(b, _):
            o = b * 256
            for k in range(8):
                v0 = ibuf[s][pl.ds(o + k * 16, L)]
                v1 = ibuf[s][pl.ds(o + 128 + k * 16, L)]
                tgt = oidx0 + (o + k * 32)
                plsc.store_scatter(obuf[s], [tgt], v0)
                plsc.store_scatter(obuf[s], [tgt + 1], v1)
            return 0

        lax.fori_loop(0, RCH // 256, blk, 0, unroll=False)

    stage_in(jnp.int32(0), 0)

    def chunk(i2, _):
        for half in range(2):
            ci = 2 * i2 + half
            nxt = jnp.minimum(ci + 1, NRCH - 1)
            stage_in(nxt, (half + 1) % 2)
            pltpu.make_async_copy(
                tn_ref.at[pl.ds(wbase, RCH)], ibuf[half], semi[half]).wait()
            @pl.when(ci >= 2)
            def _():
                pltpu.make_async_copy(
                    obuf[half], t8_ref.at[pl.ds(wbase, RCH)],
                    semo[half]).wait()
            interleave(half)
            pltpu.async_copy(
                obuf[half], t8_ref.at[pl.ds(wbase + ci * RCH, RCH)],
                semo[half])
        return 0

    lax.fori_loop(0, NRCH // 2, chunk, 0, unroll=False)
    pltpu.make_async_copy(tn_ref.at[pl.ds(wbase, RCH)], ibuf[0], semi[0]).wait()
    pltpu.make_async_copy(obuf[0], t8_ref.at[pl.ds(wbase, RCH)], semo[0]).wait()
    pltpu.make_async_copy(obuf[1], t8_ref.at[pl.ds(wbase, RCH)], semo[1]).wait()


def _phase_a(xs, idx_s, lo_s, w_s, st):
    def group(g, _):
        sl = pl.ds(g * L, L)
        px = xs[0, sl]
        py = xs[1, sl]
        pz = xs[2, sl]
        for lv in range(LPS):
            lvl = st * LPS + lv
            res = _RES[lvl]
            rf = jnp.float32(res)
            pox, poy, poz = px * rf, py * rf, pz * rf
            ix = pox.astype(jnp.int32)
            iy = poy.astype(jnp.int32)
            iz = poz.astype(jnp.int32)
            fx = pox - ix.astype(jnp.float32)
            fy = poy - iy.astype(jnp.float32)
            fz = poz - iz.astype(jnp.float32)
            if _DENSE[lvl]:
                stride = jnp.int32(res + 1)
                stride2 = jnp.int32((res + 1) * (res + 1))
                ax0, ax1 = ix, ix + 1
                by0 = iy * stride
                by1 = by0 + stride
                cz0 = iz * stride2 + jnp.int32(lvl * T)
                cz1 = cz0 + stride2
                xy = [ax0 + by0, ax1 + by0, ax0 + by1, ax1 + by1]
                for c in range(8):
                    idx = xy[c & 3] + (cz0 if c < 4 else cz1)
                    r = lv * 8 + c
                    idx_s[r, sl] = idx >> 2
                    lo_s[r, sl] = (idx & 3) * 2
            else:
                ax0, ax1 = ix, ix + 1
                by0 = iy * _P1
                by1 = by0 + _P1
                cz0 = iz * _P2
                cz1 = cz0 + _P2
                off4 = jnp.int32(lvl * (T // 4))
                m4 = jnp.int32((T - 1) >> 2)
                xy = [ax0 ^ by0, ax1 ^ by0, ax0 ^ by1, ax1 ^ by1]
                for c in range(8):
                    h = xy[c & 3] ^ (cz0 if c < 4 else cz1)
                    r = lv * 8 + c
                    idx_s[r, sl] = ((h >> 2) & m4) + off4
                    lo_s[r, sl] = (h & 3) * 2
            wx1, wy1, wz1 = fx, fy, fz
            wx0, wy0, wz0 = 1.0 - fx, 1.0 - fy, 1.0 - fz
            wxy = [wx0 * wy0, wx1 * wy0, wx0 * wy1, wx1 * wy1]
            for c in range(8):
                w_s[lv * 8 + c, sl] = wxy[c & 3] * (wz0 if c < 4 else wz1)
        return 0

    lax.fori_loop(0, C // L, group, 0, unroll=False)


def _fire(t_ref, idx_s, rows_s, sem):
    def body(j, _):
        pltpu.async_copy(t_ref.at[idx_s.at[j]], rows_s.at[j], sem)
        return 0

    lax.fori_loop(0, NSTREAM, body, 0, unroll=False)


def _drain(t_ref, idx_s, rows_s, sem):
    def body(j, _):
        pltpu.make_async_copy(t_ref.at[idx_s.at[j]], rows_s.at[j], sem).wait()
        return 0

    lax.fori_loop(0, NSTREAM, body, 0, unroll=False)


def _phase_b(lo_s, w_s, rows_s, out_v, st):
    def group(g, _):
        ii = _iota16()
        pt = g * L + ii
        sl = pl.ds(g * L, L)
        for lv in range(LPS):
            acc0 = None
            acc1 = None
            for c in range(8):
                j = lv * 8 + c
                jv = jnp.full((L,), j, jnp.int32)
                wv = w_s[j, sl]
                lo = lo_s[j, sl]
                v0 = plsc.load_gather(rows_s, [jv, pt, lo])
                v1 = plsc.load_gather(rows_s, [jv, pt, lo + 1])
                if acc0 is None:
                    acc0, acc1 = wv * v0, wv * v1
                else:
                    acc0 = acc0 + wv * v0
                    acc1 = acc1 + wv * v1
            out_v[st, pl.ds(2 * lv * 128 + g * L, L)] = acc0
            out_v[st, pl.ds(2 * lv * 128 + 128 + g * L, L)] = acc1
        return 0

    lax.fori_loop(0, C // L, group, 0, unroll=False)


def _body(x_ref, t_ref, out_ref,
          xs0, xs1, idx0, idx1, lo0, lo1, w0, w1, rows0, rows1, out_v,
          sem0, sem1, semo):
    wid = lax.axis_index("c") * NS + lax.axis_index("s")
    pbase = wid * PW

    xs = (xs0, xs1)
    idx = (idx0, idx1)
    lo = (lo0, lo1)
    wb = (w0, w1)
    rows = (rows0, rows1)
    sem = (sem0, sem1)

    def fire_unit(ci, st, s):
        base = pbase + ci * C
        pltpu.sync_copy(x_ref.at[:, pl.ds(base, C)], xs[s])
        _phase_a(xs[s], idx[s], lo[s], wb[s], st)
        _fire(t_ref, idx[s], rows[s], sem[s])

    def out_dst(i):
        return out_ref.at[:, pl.ds((wid * NCHUNK + i) * 1024, 1024)]

    fire_unit(jnp.int32(0), 0, 0)

    def chunk(i, _):
        for st in range(NSET):
            if st < NSET - 1:
                fire_unit(i, st + 1, (st + 1) % 2)
            else:
                fire_unit(jnp.minimum(i + 1, NCHUNK - 1), 0, 0)
            _drain(t_ref, idx[st % 2], rows[st % 2], sem[st % 2])
            if st == 0:
                @pl.when(i >= 1)
                def _():
                    pltpu.make_async_copy(out_v, out_dst(i - 1), semo).wait()
            _phase_b(lo[st % 2], wb[st % 2], rows[st % 2], out_v, st)
        pltpu.async_copy(out_v, out_dst(i), semo)
        return 0

    lax.fori_loop(0, NCHUNK, chunk, 0, unroll=False)
    _drain(t_ref, idx[0], rows[0], sem[0])
    pltpu.make_async_copy(out_v, out_dst(NCHUNK - 1), semo).wait()


@jax.jit
def _grid_encode(xt, tn):
    mesh = plsc.VectorSubcoreMesh(core_axis_name="c", subcore_axis_name="s")
    cp = pltpu.CompilerParams(needs_layout_passes=False,
                              use_tc_tiling_on_sc=False)

    relayout = pl.kernel(
        _relayout_body,
        out_type=jax.ShapeDtypeStruct((TW,), jnp.float32),
        mesh=mesh,
        scratch_types=[
            pltpu.VMEM((RCH,), jnp.float32),
            pltpu.VMEM((RCH,), jnp.float32),
            pltpu.VMEM((RCH,), jnp.float32),
            pltpu.VMEM((RCH,), jnp.float32),
            pltpu.SemaphoreType.DMA,
            pltpu.SemaphoreType.DMA,
            pltpu.SemaphoreType.DMA,
            pltpu.SemaphoreType.DMA,
        ],
        compiler_params=cp,
    )
    t8 = relayout(tn).reshape(NROWS8, 8)

    scratch = [
        pltpu.VMEM((3, C), jnp.float32),
        pltpu.VMEM((3, C), jnp.float32),
        pltpu.VMEM((NSTREAM, C), jnp.int32),
        pltpu.VMEM((NSTREAM, C), jnp.int32),
        pltpu.VMEM((NSTREAM, C), jnp.int32),
        pltpu.VMEM((NSTREAM, C), jnp.int32),
        pltpu.VMEM((NSTREAM, C), jnp.float32),
        pltpu.VMEM((NSTREAM, C), jnp.float32),
        pltpu.VMEM((NSTREAM, C, 8), jnp.float32),
        pltpu.VMEM((NSTREAM, C, 8), jnp.float32),
        pltpu.VMEM((4, 1024), jnp.float32),
        pltpu.SemaphoreType.DMA,
        pltpu.SemaphoreType.DMA,
        pltpu.SemaphoreType.DMA,
    ]
    run = pl.kernel(
        _body,
        out_type=jax.ShapeDtypeStruct((4, N_POINTS * 8), jnp.float32),
        mesh=mesh,
        scratch_types=scratch,
        compiler_params=cp,
    )
    return run(xt, t8)


def kernel(x, table):
    tn = (table.reshape(N_LEVELS, T // 128, 128, F)
          .transpose(0, 1, 3, 2)
          .reshape(TW))
    out = _grid_encode(x.T, tn)
    return (out.reshape(4, N_POINTS // 128, 8, 128)
            .transpose(1, 3, 0, 2)
            .reshape(N_POINTS, 2 * N_LEVELS))


# speedup vs baseline: 1.7668x; 1.0003x over previous
import jax
import jax.numpy as jnp
import numpy as np
from jax import lax
from jax.experimental import pallas as pl
from jax.experimental.pallas import tpu as pltpu
from jax.experimental.pallas import tpu_sc as plsc

N_POINTS = 524288
DIM = 3
N_LEVELS = 16
F = 2
LOG2_T = 19
T = 2 ** LOG2_T
BASE_RES = 16
PER_LEVEL_SCALE = 1.5

NC = 2
NS = 16
NW = NC * NS
L = 16

PW = N_POINTS // NW
C = 128
NCHUNK = PW // C
LPS = 4
NSET = N_LEVELS // LPS
NSTREAM = LPS * 8
TW = N_LEVELS * T * F
NROWS8 = TW // 8

WPW = TW // NW
RCH = 16384
NRCH = WPW // RCH

_P1 = np.int32(np.uint32(2654435761).astype(np.int32))
_P2 = np.int32(np.uint32(805459861).astype(np.int32))
_MASK = np.int32(T - 1)

_RES = [int(np.floor(BASE_RES * (PER_LEVEL_SCALE ** l))) for l in range(N_LEVELS)]
_DENSE = [(r + 1) ** DIM <= T for r in _RES]


def _iota16():
    return lax.broadcasted_iota(jnp.int32, (L,), 0)


def _relayout_body(tn_ref, t8_ref, ibuf0, ibuf1, obuf0, obuf1,
                   semi0, semi1, semo0, semo1):
    wid = lax.axis_index("c") * NS + lax.axis_index("s")
    wbase = wid * WPW

    ibuf = (ibuf0, ibuf1)
    obuf = (obuf0, obuf1)
    semi = (semi0, semi1)
    semo = (semo0, semo1)

    def stage_in(ci, s):
        pltpu.async_copy(tn_ref.at[pl.ds(wbase + ci * RCH, RCH)],
                         ibuf[s], semi[s])

    oidx0 = 2 * _iota16()

    def interleave(s):
        def blk(b, _):
            o = b * 256
            for k in range(8):
                v0 = ibuf[s][pl.ds(o + k * 16, L)]
                v1 = ibuf[s][pl.ds(o + 128 + k * 16, L)]
                tgt = oidx0 + (o + k * 32)
                plsc.store_scatter(obuf[s], [tgt], v0)
                plsc.store_scatter(obuf[s], [tgt + 1], v1)
            return 0

        lax.fori_loop(0, RCH // 256, blk, 0, unroll=False)

    stage_in(jnp.int32(0), 0)

    def chunk(i2, _):
        for half in range(2):
            ci = 2 * i2 + half
            nxt = jnp.minimum(ci + 1, NRCH - 1)
            stage_in(nxt, (half + 1) % 2)
            pltpu.make_async_copy(
                tn_ref.at[pl.ds(wbase, RCH)], ibuf[half], semi[half]).wait()
            @pl.when(ci >= 2)
            def _():
                pltpu.make_async_copy(
                    obuf[half], t8_ref.at[pl.ds(wbase, RCH)],
                    semo[half]).wait()
            interleave(half)
            pltpu.async_copy(
                obuf[half], t8_ref.at[pl.ds(wbase + ci * RCH, RCH)],
                semo[half])
        return 0

    lax.fori_loop(0, NRCH // 2, chunk, 0, unroll=False)
    pltpu.make_async_copy(tn_ref.at[pl.ds(wbase, RCH)], ibuf[0], semi[0]).wait()
    pltpu.make_async_copy(obuf[0], t8_ref.at[pl.ds(wbase, RCH)], semo[0]).wait()
    pltpu.make_async_copy(obuf[1], t8_ref.at[pl.ds(wbase, RCH)], semo[1]).wait()


def _phase_a(xs, idx_s, lo_s, w_s, st):
    def group(g, _):
        sl = pl.ds(g * L, L)
        px = xs[0, sl]
        py = xs[1, sl]
        pz = xs[2, sl]
        for lv in range(LPS):
            lvl = st * LPS + lv
            res = _RES[lvl]
            rf = jnp.float32(res)
            pox, poy, poz = px * rf, py * rf, pz * rf
            ix = pox.astype(jnp.int32)
            iy = poy.astype(jnp.int32)
            iz = poz.astype(jnp.int32)
            fx = pox - ix.astype(jnp.float32)
            fy = poy - iy.astype(jnp.float32)
            fz = poz - iz.astype(jnp.float32)
            if _DENSE[lvl]:
                stride = jnp.int32(res + 1)
                stride2 = jnp.int32((res + 1) * (res + 1))
                ax0, ax1 = ix, ix + 1
                by0 = iy * stride
                by1 = by0 + stride
                cz0 = iz * stride2 + jnp.int32(lvl * T)
                cz1 = cz0 + stride2
                xy = [ax0 + by0, ax1 + by0, ax0 + by1, ax1 + by1]
                for c in range(8):
                    idx = xy[c & 3] + (cz0 if c < 4 else cz1)
                    r = lv * 8 + c
                    idx_s[r, sl] = idx >> 2
                    lo_s[r, sl] = (idx & 3) * 2
            else:
                ax0, ax1 = ix, ix + 1
                by0 = iy * _P1
                by1 = by0 + _P1
                cz0 = iz * _P2
                cz1 = cz0 + _P2
                off4 = jnp.int32(lvl * (T // 4))
                m4 = jnp.int32((T - 1) >> 2)
                xy = [ax0 ^ by0, ax1 ^ by0, ax0 ^ by1, ax1 ^ by1]
                for c in range(8):
                    h = xy[c & 3] ^ (cz0 if c < 4 else cz1)
                    r = lv * 8 + c
                    idx_s[r, sl] = ((h >> 2) & m4) + off4
                    lo_s[r, sl] = (h & 3) * 2
            wx1, wy1, wz1 = fx, fy, fz
            wx0, wy0, wz0 = 1.0 - fx, 1.0 - fy, 1.0 - fz
            wxy = [wx0 * wy0, wx1 * wy0, wx0 * wy1, wx1 * wy1]
            for c in range(8):
                w_s[lv * 8 + c, sl] = wxy[c & 3] * (wz0 if c < 4 else wz1)
        return 0

    lax.fori_loop(0, C // L, group, 0, unroll=2)


def _fire(t_ref, idx_s, rows_s, sem):
    def body(j, _):
        pltpu.async_copy(t_ref.at[idx_s.at[j]], rows_s.at[j], sem)
        return 0

    lax.fori_loop(0, NSTREAM, body, 0, unroll=False)


def _drain(t_ref, idx_s, rows_s, sem):
    def body(j, _):
        pltpu.make_async_copy(t_ref.at[idx_s.at[j]], rows_s.at[j], sem).wait()
        return 0

    lax.fori_loop(0, NSTREAM, body, 0, unroll=False)


def _phase_b(lo_s, w_s, rows_s, out_v, st):
    def group(g, _):
        ii = _iota16()
        pt = g * L + ii
        sl = pl.ds(g * L, L)
        for lv in range(LPS):
            acc0 = None
            acc1 = None
            for c in range(8):
                j = lv * 8 + c
                jv = jnp.full((L,), j, jnp.int32)
                wv = w_s[j, sl]
                lo = lo_s[j, sl]
                v0 = plsc.load_gather(rows_s, [jv, pt, lo])
                v1 = plsc.load_gather(rows_s, [jv, pt, lo + 1])
                if acc0 is None:
                    acc0, acc1 = wv * v0, wv * v1
                else:
                    acc0 = acc0 + wv * v0
                    acc1 = acc1 + wv * v1
            out_v[st, pl.ds(2 * lv * 128 + g * L, L)] = acc0
            out_v[st, pl.ds(2 * lv * 128 + 128 + g * L, L)] = acc1
        return 0

    lax.fori_loop(0, C // L, group, 0, unroll=2)


def _body(x_ref, t_ref, out_ref,
          xs0, xs1, idx0, idx1, lo0, lo1, w0, w1, rows0, rows1, out_v,
          sem0, sem1, semo):
    wid = lax.axis_index("c") * NS + lax.axis_index("s")
    pbase = wid * PW

    xs = (xs0, xs1)
    idx = (idx0, idx1)
    lo = (lo0, lo1)
    wb = (w0, w1)
    rows = (rows0, rows1)
    sem = (sem0, sem1)

    def fire_unit(ci, st, s):
        base = pbase + ci * C
        pltpu.sync_copy(x_ref.at[:, pl.ds(base, C)], xs[s])
        _phase_a(xs[s], idx[s], lo[s], wb[s], st)
        _fire(t_ref, idx[s], rows[s], sem[s])

    def out_dst(i):
        return out_ref.at[:, pl.ds((wid * NCHUNK + i) * 1024, 1024)]

    fire_unit(jnp.int32(0), 0, 0)

    def chunk(i, _):
        for st in range(NSET):
            if st < NSET - 1:
                fire_unit(i, st + 1, (st + 1) % 2)
            else:
                fire_unit(jnp.minimum(i + 1, NCHUNK - 1), 0, 0)
            _drain(t_ref, idx[st % 2], rows[st % 2], sem[st % 2])
            if st == 0:
                @pl.when(i >= 1)
                def _():
                    pltpu.make_async_copy(out_v, out_dst(i - 1), semo).wait()
            _phase_b(lo[st % 2], wb[st % 2], rows[st % 2], out_v, st)
        pltpu.async_copy(out_v, out_dst(i), semo)
        return 0

    lax.fori_loop(0, NCHUNK, chunk, 0, unroll=False)
    _drain(t_ref, idx[0], rows[0], sem[0])
    pltpu.make_async_copy(out_v, out_dst(NCHUNK - 1), semo).wait()


@jax.jit
def _grid_encode(xt, tn):
    mesh = plsc.VectorSubcoreMesh(core_axis_name="c", subcore_axis_name="s")
    cp = pltpu.CompilerParams(needs_layout_passes=False,
                              use_tc_tiling_on_sc=False)

    relayout = pl.kernel(
        _relayout_body,
        out_type=jax.ShapeDtypeStruct((TW,), jnp.float32),
        mesh=mesh,
        scratch_types=[
            pltpu.VMEM((RCH,), jnp.float32),
            pltpu.VMEM((RCH,), jnp.float32),
            pltpu.VMEM((RCH,), jnp.float32),
            pltpu.VMEM((RCH,), jnp.float32),
            pltpu.SemaphoreType.DMA,
            pltpu.SemaphoreType.DMA,
            pltpu.SemaphoreType.DMA,
            pltpu.SemaphoreType.DMA,
        ],
        compiler_params=cp,
    )
    t8 = relayout(tn).reshape(NROWS8, 8)

    scratch = [
        pltpu.VMEM((3, C), jnp.float32),
        pltpu.VMEM((3, C), jnp.float32),
        pltpu.VMEM((NSTREAM, C), jnp.int32),
        pltpu.VMEM((NSTREAM, C), jnp.int32),
        pltpu.VMEM((NSTREAM, C), jnp.int32),
        pltpu.VMEM((NSTREAM, C), jnp.int32),
        pltpu.VMEM((NSTREAM, C), jnp.float32),
        pltpu.VMEM((NSTREAM, C), jnp.float32),
        pltpu.VMEM((NSTREAM, C, 8), jnp.float32),
        pltpu.VMEM((NSTREAM, C, 8), jnp.float32),
        pltpu.VMEM((4, 1024), jnp.float32),
        pltpu.SemaphoreType.DMA,
        pltpu.SemaphoreType.DMA,
        pltpu.SemaphoreType.DMA,
    ]
    run = pl.kernel(
        _body,
        out_type=jax.ShapeDtypeStruct((4, N_POINTS * 8), jnp.float32),
        mesh=mesh,
        scratch_types=scratch,
        compiler_params=cp,
    )
    return run(xt, t8)


def kernel(x, table):
    tn = (table.reshape(N_LEVELS, T // 128, 128, F)
          .transpose(0, 1, 3, 2)
          .reshape(TW))
    out = _grid_encode(x.T, tn)
    return (out.reshape(4, N_POINTS // 128, 8, 128)
            .transpose(1, 3, 0, 2)
            .reshape(N_POINTS, 2 * N_LEVELS))


# speedup vs baseline: 1.7669x; 1.0000x over previous
import jax
import jax.numpy as jnp
import numpy as np
from jax import lax
from jax.experimental import pallas as pl
from jax.experimental.pallas import tpu as pltpu
from jax.experimental.pallas import tpu_sc as plsc

N_POINTS = 524288
DIM = 3
N_LEVELS = 16
F = 2
LOG2_T = 19
T = 2 ** LOG2_T
BASE_RES = 16
PER_LEVEL_SCALE = 1.5

NC = 2
NS = 16
NW = NC * NS
L = 16

PW = N_POINTS // NW
C = 128
NCHUNK = PW // C
LPS = 4
NSET = N_LEVELS // LPS
NSTREAM = LPS * 8
TW = N_LEVELS * T * F
NROWS8 = TW // 8

WPW = TW // NW
RCH = 16384
NRCH = WPW // RCH

_P1 = np.int32(np.uint32(2654435761).astype(np.int32))
_P2 = np.int32(np.uint32(805459861).astype(np.int32))
_MASK = np.int32(T - 1)

_RES = [int(np.floor(BASE_RES * (PER_LEVEL_SCALE ** l))) for l in range(N_LEVELS)]
_DENSE = [(r + 1) ** DIM <= T for r in _RES]


def _iota16():
    return lax.broadcasted_iota(jnp.int32, (L,), 0)


def _relayout_body(tn_ref, t8_ref, ibuf0, ibuf1, obuf0, obuf1,
                   semi0, semi1, semo0, semo1):
    wid = lax.axis_index("c") * NS + lax.axis_index("s")
    wbase = wid * WPW

    ibuf = (ibuf0, ibuf1)
    obuf = (obuf0, obuf1)
    semi = (semi0, semi1)
    semo = (semo0, semo1)

    def stage_in(ci, s):
        pltpu.async_copy(tn_ref.at[pl.ds(wbase + ci * RCH, RCH)],
                         ibuf[s], semi[s])

    oidx0 = 2 * _iota16()

    def interleave(s):
        def blk(b, _):
            o = b * 256
            for k in range(8):
                v0 = ibuf[s][pl.ds(o + k * 16, L)]
                v1 = ibuf[s][pl.ds(o + 128 + k * 16, L)]
                tgt = oidx0 + (o + k * 32)
                plsc.store_scatter(obuf[s], [tgt], v0)
                plsc.store_scatter(obuf[s], [tgt + 1], v1)
            return 0

        lax.fori_loop(0, RCH // 256, blk, 0, unroll=False)

    stage_in(jnp.int32(0), 0)

    def chunk(i2, _):
        for half in range(2):
            ci = 2 * i2 + half
            nxt = jnp.minimum(ci + 1, NRCH - 1)
            stage_in(nxt, (half + 1) % 2)
            pltpu.make_async_copy(
                tn_ref.at[pl.ds(wbase, RCH)], ibuf[half], semi[half]).wait()
            @pl.when(ci >= 2)
            def _():
                pltpu.make_async_copy(
                    obuf[half], t8_ref.at[pl.ds(wbase, RCH)],
                    semo[half]).wait()
            interleave(half)
            pltpu.async_copy(
                obuf[half], t8_ref.at[pl.ds(wbase + ci * RCH, RCH)],
                semo[half])
        return 0

    lax.fori_loop(0, NRCH // 2, chunk, 0, unroll=False)
    pltpu.make_async_copy(tn_ref.at[pl.ds(wbase, RCH)], ibuf[0], semi[0]).wait()
    pltpu.make_async_copy(obuf[0], t8_ref.at[pl.ds(wbase, RCH)], semo[0]).wait()
    pltpu.make_async_copy(obuf[1], t8_ref.at[pl.ds(wbase, RCH)], semo[1]).wait()


def _phase_a(xs, idx_s, lo_s, w_s, st):
    def group(g, _):
        sl = pl.ds(g * L, L)
        px = xs[0, sl]
        py = xs[1, sl]
        pz = xs[2, sl]
        p8 = _iota16() * 8 + g * (L * 8)
        for lv in range(LPS):
            lvl = st * LPS + lv
            res = _RES[lvl]
            rf = jnp.float32(res)
            pox, poy, poz = px * rf, py * rf, pz * rf
            ix = pox.astype(jnp.int32)
            iy = poy.astype(jnp.int32)
            iz = poz.astype(jnp.int32)
            fx = pox - ix.astype(jnp.float32)
            fy = poy - iy.astype(jnp.float32)
            fz = poz - iz.astype(jnp.float32)
            if _DENSE[lvl]:
                stride = jnp.int32(res + 1)
                stride2 = jnp.int32((res + 1) * (res + 1))
                ax0, ax1 = ix, ix + 1
                by0 = iy * stride
                by1 = by0 + stride
                cz0 = iz * stride2 + jnp.int32(lvl * T)
                cz1 = cz0 + stride2
                xy = [ax0 + by0, ax1 + by0, ax0 + by1, ax1 + by1]
                for c in range(8):
                    idx = xy[c & 3] + (cz0 if c < 4 else cz1)
                    r = lv * 8 + c
                    idx_s[r, sl] = idx >> 2
                    lo_s[r, sl] = (idx & 3) * 2 + p8
            else:
                ax0, ax1 = ix, ix + 1
                by0 = iy * _P1
                by1 = by0 + _P1
                cz0 = iz * _P2
                cz1 = cz0 + _P2
                off4 = jnp.int32(lvl * (T // 4))
                m4 = jnp.int32((T - 1) >> 2)
                xy = [ax0 ^ by0, ax1 ^ by0, ax0 ^ by1, ax1 ^ by1]
                for c in range(8):
                    h = xy[c & 3] ^ (cz0 if c < 4 else cz1)
                    r = lv * 8 + c
                    idx_s[r, sl] = ((h >> 2) & m4) + off4
                    lo_s[r, sl] = (h & 3) * 2 + p8
            wx1, wy1, wz1 = fx, fy, fz
            wx0, wy0, wz0 = 1.0 - fx, 1.0 - fy, 1.0 - fz
            wxy = [wx0 * wy0, wx1 * wy0, wx0 * wy1, wx1 * wy1]
            for c in range(8):
                w_s[lv * 8 + c, sl] = wxy[c & 3] * (wz0 if c < 4 else wz1)
        return 0

    lax.fori_loop(0, C // L, group, 0, unroll=2)


def _fire(t_ref, idx_s, rows_s, sem):
    def body(j, _):
        pltpu.async_copy(t_ref.at[idx_s.at[j]], rows_s.at[j], sem)
        return 0

    lax.fori_loop(0, NSTREAM, body, 0, unroll=False)


def _drain(t_ref, idx_s, rows_s, sem):
    def body(j, _):
        pltpu.make_async_copy(t_ref.at[idx_s.at[j]], rows_s.at[j], sem).wait()
        return 0

    lax.fori_loop(0, NSTREAM, body, 0, unroll=False)


def _phase_b(lo_s, w_s, rows_s, out_v, st):
    def group(g, _):
        zero = jnp.zeros((L,), jnp.int32)
        sl = pl.ds(g * L, L)
        for lv in range(LPS):
            acc0 = None
            acc1 = None
            for c in range(8):
                j = lv * 8 + c
                jv = jnp.full((L,), j, jnp.int32)
                wv = w_s[j, sl]
                lo = lo_s[j, sl]
                v0 = plsc.load_gather(rows_s, [jv, zero, lo])
                v1 = plsc.load_gather(rows_s, [jv, zero, lo + 1])
                if acc0 is None:
                    acc0, acc1 = wv * v0, wv * v1
                else:
                    acc0 = acc0 + wv * v0
                    acc1 = acc1 + wv * v1
            out_v[st, pl.ds(2 * lv * 128 + g * L, L)] = acc0
            out_v[st, pl.ds(2 * lv * 128 + 128 + g * L, L)] = acc1
        return 0

    lax.fori_loop(0, C // L, group, 0, unroll=2)


def _body(x_ref, t_ref, out_ref,
          xs0, xs1, idx0, idx1, lo0, lo1, w0, w1, rows0, rows1, out_v,
          sem0, sem1, semo):
    wid = lax.axis_index("c") * NS + lax.axis_index("s")
    pbase = wid * PW

    xs = (xs0, xs1)
    idx = (idx0, idx1)
    lo = (lo0, lo1)
    wb = (w0, w1)
    rows = (rows0, rows1)
    sem = (sem0, sem1)

    def fire_unit(ci, st, s):
        base = pbase + ci * C
        pltpu.sync_copy(x_ref.at[:, pl.ds(base, C)], xs[s])
        _phase_a(xs[s], idx[s], lo[s], wb[s], st)
        _fire(t_ref, idx[s], rows[s], sem[s])

    def out_dst(i):
        return out_ref.at[:, pl.ds((wid * NCHUNK + i) * 1024, 1024)]

    fire_unit(jnp.int32(0), 0, 0)

    def chunk(i, _):
        for st in range(NSET):
            if st < NSET - 1:
                fire_unit(i, st + 1, (st + 1) % 2)
            else:
                fire_unit(jnp.minimum(i + 1, NCHUNK - 1), 0, 0)
            _drain(t_ref, idx[st % 2], rows[st % 2], sem[st % 2])
            if st == 0:
                @pl.when(i >= 1)
                def _():
                    pltpu.make_async_copy(out_v, out_dst(i - 1), semo).wait()
            _phase_b(lo[st % 2], wb[st % 2], rows[st % 2], out_v, st)
        pltpu.async_copy(out_v, out_dst(i), semo)
        return 0

    lax.fori_loop(0, NCHUNK, chunk, 0, unroll=False)
    _drain(t_ref, idx[0], rows[0], sem[0])
    pltpu.make_async_copy(out_v, out_dst(NCHUNK - 1), semo).wait()


@jax.jit
def _grid_encode(xt, tn):
    mesh = plsc.VectorSubcoreMesh(core_axis_name="c", subcore_axis_name="s")
    cp = pltpu.CompilerParams(needs_layout_passes=False,
                              use_tc_tiling_on_sc=False)

    relayout = pl.kernel(
        _relayout_body,
        out_type=jax.ShapeDtypeStruct((TW,), jnp.float32),
        mesh=mesh,
        scratch_types=[
            pltpu.VMEM((RCH,), jnp.float32),
            pltpu.VMEM((RCH,), jnp.float32),
            pltpu.VMEM((RCH,), jnp.float32),
            pltpu.VMEM((RCH,), jnp.float32),
            pltpu.SemaphoreType.DMA,
            pltpu.SemaphoreType.DMA,
            pltpu.SemaphoreType.DMA,
            pltpu.SemaphoreType.DMA,
        ],
        compiler_params=cp,
    )
    t8 = relayout(tn).reshape(NROWS8, 8)

    scratch = [
        pltpu.VMEM((3, C), jnp.float32),
        pltpu.VMEM((3, C), jnp.float32),
        pltpu.VMEM((NSTREAM, C), jnp.int32),
        pltpu.VMEM((NSTREAM, C), jnp.int32),
        pltpu.VMEM((NSTREAM, C), jnp.int32),
        pltpu.VMEM((NSTREAM, C), jnp.int32),
        pltpu.VMEM((NSTREAM, C), jnp.float32),
        pltpu.VMEM((NSTREAM, C), jnp.float32),
        pltpu.VMEM((NSTREAM, C, 8), jnp.float32),
        pltpu.VMEM((NSTREAM, C, 8), jnp.float32),
        pltpu.VMEM((4, 1024), jnp.float32),
        pltpu.SemaphoreType.DMA,
        pltpu.SemaphoreType.DMA,
        pltpu.SemaphoreType.DMA,
    ]
    run = pl.kernel(
        _body,
        out_type=jax.ShapeDtypeStruct((4, N_POINTS * 8), jnp.float32),
        mesh=mesh,
        scratch_types=scratch,
        compiler_params=cp,
    )
    return run(xt, t8)


def kernel(x, table):
    tn = (table.reshape(N_LEVELS, T // 128, 128, F)
          .transpose(0, 1, 3, 2)
          .reshape(TW))
    out = _grid_encode(x.T, tn)
    return (out.reshape(4, N_POINTS // 128, 8, 128)
            .transpose(1, 3, 0, 2)
            .reshape(N_POINTS, 2 * N_LEVELS))
